# baseline (speedup 1.0000x reference)
import jax
import jax.numpy as jnp
from jax import lax
from jax.experimental import pallas as pl
from jax.experimental.pallas import tpu as pltpu
from jax.experimental.pallas import tpu_sc as plsc

_B, _N, _D = 64, 576, 768
_NC, _NS = 2, 16
_NG = 4
_NR = 8
_RPW = _N // _NR
_BPW = _B // _NG
_SUB = 24
_SPB = _RPW // _SUB
_STEPS = _BPW * _SPB
_NBUF = 4
_LANES = 16
_COLS = _D // _LANES


def _sc_body(x_hbm, p_hbm, o_hbm, pos_v, bufs, s0, s1, s2, s3, t0, t1, t2, t3):
    insems = (s0, s1, s2, s3)
    outsems = (t0, t1, t2, t3)
    wid = lax.axis_index("s") * _NC + lax.axis_index("c")
    g = wid // _NR
    i = wid % _NR
    p0 = i * _RPW
    b0 = g * _BPW

    pltpu.sync_copy(p_hbm.at[pl.ds(p0, _RPW)], pos_v)

    def step_loc(t):
        return b0 + t // _SPB, p0 + (t % _SPB) * _SUB

    def in_start(j, t):
        b, r0 = step_loc(t)
        pltpu.async_copy(x_hbm.at[b, pl.ds(r0, _SUB)], bufs.at[j], insems[j])

    def in_wait(j):
        pltpu.make_async_copy(
            x_hbm.at[0, pl.ds(0, _SUB)], bufs.at[j], insems[j]).wait()

    def out_start(j, t):
        b, r0 = step_loc(t)
        pltpu.async_copy(bufs.at[j], o_hbm.at[b, pl.ds(r0, _SUB)], outsems[j])

    def out_wait(j):
        pltpu.make_async_copy(
            bufs.at[j], o_hbm.at[0, pl.ds(0, _SUB)], outsems[j]).wait()

    def compute(j, t):
        rbase = (t % _SPB) * _SUB

        def row_body(r, acc):
            for c in range(_COLS):
                cs = pl.ds(c * _LANES, _LANES)
                plsc.addupdate(bufs.at[j, r, cs], pos_v[rbase + r, cs])
            return acc

        lax.fori_loop(0, _SUB, row_body, 0)

    in_start(0, 0)
    in_start(1, 1)

    def outer(tt, acc):
        for j in range(_NBUF):
            t = tt + j
            in_wait(j)
            compute(j, t)
            out_start(j, t)
            j2 = (j + 2) % _NBUF

            @pl.when(t + 2 < _STEPS)
            def _():
                @pl.when(t >= 2)
                def _():
                    out_wait(j2)
                in_start(j2, t + 2)
        return acc

    lax.fori_loop(0, _STEPS // _NBUF, lambda q, a: outer(q * _NBUF, a), 0)
    for j in range(_NBUF):
        out_wait(j)


def kernel(inputs, pos_table):
    mesh = plsc.VectorSubcoreMesh(core_axis_name="c", subcore_axis_name="s")
    f = pl.kernel(
        _sc_body,
        out_type=jax.ShapeDtypeStruct((_B, _N, _D), jnp.float32),
        mesh=mesh,
        scratch_types=[
            pltpu.VMEM((_RPW, _D), jnp.float32),
            pltpu.VMEM((_NBUF, _SUB, _D), jnp.float32),
        ] + [pltpu.SemaphoreType.DMA] * (2 * _NBUF),
    )
    return f(inputs, pos_table)


# speedup vs baseline: 1.5392x; 1.5392x over previous
import jax
import jax.numpy as jnp
from jax import lax
from jax.experimental import pallas as pl
from jax.experimental.pallas import tpu as pltpu
from jax.experimental.pallas import tpu_sc as plsc

_B, _N, _D = 64, 576, 768
_NC, _NS = 2, 16
_NG = 4
_NR = 8
_RPW = _N // _NR
_BPW = _B // _NG
_SUB = 24
_SPB = _RPW // _SUB
_NBUF = 4
_LANES = 16
_COLS = _D // _LANES


def _sc_body(x_hbm, p_hbm, o_hbm, pos_v, bufs, s0, s1, s2, s3, t0, t1, t2, t3):
    insems = (s0, s1, s2, s3)
    outsems = (t0, t1, t2, t3)
    wid = lax.axis_index("s") * _NC + lax.axis_index("c")
    g = wid // _NR
    i = wid % _NR
    p0 = i * _RPW
    b0 = g * _BPW

    def in_start(j, b, r0):
        pltpu.async_copy(x_hbm.at[b, pl.ds(r0, _SUB)], bufs.at[j], insems[j])

    def in_wait(j):
        pltpu.make_async_copy(
            x_hbm.at[0, pl.ds(0, _SUB)], bufs.at[j], insems[j]).wait()

    def out_start(j, b, r0):
        pltpu.async_copy(bufs.at[j], o_hbm.at[b, pl.ds(r0, _SUB)], outsems[j])

    def out_wait(j):
        pltpu.make_async_copy(
            bufs.at[j], o_hbm.at[0, pl.ds(0, _SUB)], outsems[j]).wait()

    def compute(j):
        def row_body(r, acc):
            for c in range(_COLS):
                cs = pl.ds(c * _LANES, _LANES)
                plsc.addupdate(bufs.at[j, r, cs], pos_v[r, cs])
            return acc
        lax.fori_loop(0, _SUB, row_body, 0)

    for sub in range(_SPB):
        r0 = p0 + sub * _SUB
        if sub == 0:
            pltpu.sync_copy(p_hbm.at[pl.ds(r0, _SUB)], pos_v)
            in_start(0, b0, r0)
            in_start(1, b0 + 1, r0)

        def outer(tt, acc, sub=sub, r0=r0):
            for j in range(_NBUF):
                t = tt + j
                in_wait(j)
                compute(j)
                out_start(j, b0 + t, r0)
                j2 = (j + 2) % _NBUF

                if sub > 0 and j2 >= 2:
                    @pl.when(t + 2 < _BPW)
                    def _():
                        out_wait(j2)
                        in_start(j2, b0 + t + 2, r0)
                else:
                    @pl.when(t + 2 < _BPW)
                    def _():
                        @pl.when(t >= 2)
                        def _():
                            out_wait(j2)
                        in_start(j2, b0 + t + 2, r0)
            return acc

        lax.fori_loop(0, _BPW // _NBUF,
                      lambda q, a, f=outer: f(q * _NBUF, a), 0)

        if sub + 1 < _SPB:
            rn = p0 + (sub + 1) * _SUB
            pltpu.sync_copy(p_hbm.at[pl.ds(rn, _SUB)], pos_v)
            out_wait(0)
            in_start(0, b0, rn)
            out_wait(1)
            in_start(1, b0 + 1, rn)
        else:
            for j in range(_NBUF):
                out_wait(j)


def kernel(inputs, pos_table):
    mesh = plsc.VectorSubcoreMesh(core_axis_name="c", subcore_axis_name="s")
    f = pl.kernel(
        _sc_body,
        out_type=jax.ShapeDtypeStruct((_B, _N, _D), jnp.float32),
        mesh=mesh,
        scratch_types=[
            pltpu.VMEM((_SUB, _D), jnp.float32),
            pltpu.VMEM((_NBUF, _SUB, _D), jnp.float32),
        ] + [pltpu.SemaphoreType.DMA] * (2 * _NBUF),
    )
    return f(inputs, pos_table)


# speedup vs baseline: 1.9935x; 1.2951x over previous
import jax
import jax.numpy as jnp
from jax import lax
from jax.experimental import pallas as pl
from jax.experimental.pallas import tpu as pltpu
from jax.experimental.pallas import tpu_sc as plsc

_B, _N, _D = 64, 576, 768
_KSC = 16
_NC, _NS = 2, 16
_NG = 4
_NR = 8
_RPW = _N // _NR
_BPW = _KSC // _NG
_SUB = 24
_SPB = _RPW // _SUB
_NBUF = 4
_LANES = 16
_COLS = _D // _LANES


def _sc_body(x_hbm, p_hbm, o_hbm, pos_v, bufs, s0, s1, s2, s3, t0, t1, t2, t3):
    insems = (s0, s1, s2, s3)
    outsems = (t0, t1, t2, t3)
    wid = lax.axis_index("s") * _NC + lax.axis_index("c")
    g = wid // _NR
    i = wid % _NR
    p0 = i * _RPW
    b0 = g * _BPW

    def in_start(j, b, r0):
        pltpu.async_copy(x_hbm.at[b, pl.ds(r0, _SUB)], bufs.at[j], insems[j])

    def in_wait(j):
        pltpu.make_async_copy(
            x_hbm.at[0, pl.ds(0, _SUB)], bufs.at[j], insems[j]).wait()

    def out_start(j, b, r0):
        pltpu.async_copy(bufs.at[j], o_hbm.at[b, pl.ds(r0, _SUB)], outsems[j])

    def out_wait(j):
        pltpu.make_async_copy(
            bufs.at[j], o_hbm.at[0, pl.ds(0, _SUB)], outsems[j]).wait()

    def compute(j):
        def row_body(r, acc):
            for c in range(_COLS):
                cs = pl.ds(c * _LANES, _LANES)
                plsc.addupdate(bufs.at[j, r, cs], pos_v[r, cs])
            return acc
        lax.fori_loop(0, _SUB, row_body, 0)

    for sub in range(_SPB):
        r0 = p0 + sub * _SUB
        if sub == 0:
            pltpu.sync_copy(p_hbm.at[pl.ds(r0, _SUB)], pos_v)
            in_start(0, b0, r0)
            in_start(1, b0 + 1, r0)

        def outer(tt, acc, sub=sub, r0=r0):
            for j in range(_NBUF):
                t = tt + j
                in_wait(j)
                compute(j)
                out_start(j, b0 + t, r0)
                j2 = (j + 2) % _NBUF

                if sub > 0 and j2 >= 2:
                    @pl.when(t + 2 < _BPW)
                    def _():
                        out_wait(j2)
                        in_start(j2, b0 + t + 2, r0)
                else:
                    @pl.when(t + 2 < _BPW)
                    def _():
                        @pl.when(t >= 2)
                        def _():
                            out_wait(j2)
                        in_start(j2, b0 + t + 2, r0)
            return acc

        lax.fori_loop(0, _BPW // _NBUF,
                      lambda q, a, f=outer: f(q * _NBUF, a), 0)

        if sub + 1 < _SPB:
            rn = p0 + (sub + 1) * _SUB
            pltpu.sync_copy(p_hbm.at[pl.ds(rn, _SUB)], pos_v)
            out_wait(0)
            in_start(0, b0, rn)
            out_wait(1)
            in_start(1, b0 + 1, rn)
        else:
            for j in range(_NBUF):
                out_wait(j)


def _sc_call(inputs, pos_table):
    mesh = plsc.VectorSubcoreMesh(core_axis_name="c", subcore_axis_name="s")
    f = pl.kernel(
        _sc_body,
        out_type=jax.ShapeDtypeStruct((_KSC, _N, _D), jnp.float32),
        mesh=mesh,
        scratch_types=[
            pltpu.VMEM((_SUB, _D), jnp.float32),
            pltpu.VMEM((_NBUF, _SUB, _D), jnp.float32),
        ] + [pltpu.SemaphoreType.DMA] * (2 * _NBUF),
    )
    return f(inputs, pos_table)


def _tc_add_body(x_ref, p_ref, o_ref):
    o_ref[...] = x_ref[...] + p_ref[...][None]


def _tc_call(x, pos_table):
    nb = _B - _KSC
    bb = 4
    off = _KSC // bb
    return pl.pallas_call(
        _tc_add_body,
        grid=(nb // bb,),
        in_specs=[
            pl.BlockSpec((bb, _N, _D), lambda b: (b + off, 0, 0)),
            pl.BlockSpec((_N, _D), lambda b: (0, 0)),
        ],
        out_specs=pl.BlockSpec((bb, _N, _D), lambda b: (b + off, 0, 0)),
        out_shape=jax.ShapeDtypeStruct((_B, _N, _D), jnp.float32),
    )(x, pos_table)


def kernel(inputs, pos_table):
    sc_out = _sc_call(inputs, pos_table)
    tc_out = _tc_call(inputs, pos_table)
    return lax.dynamic_update_slice(tc_out, sc_out, (0, 0, 0))
